# baseline (speedup 1.0000x reference)
import functools

import jax
import jax.numpy as jnp
from jax import lax
from jax.experimental import pallas as pl
from jax.experimental.pallas import tpu as pltpu
from jax.experimental.pallas import tpu_sc as plsc

NC = 2
NS = 16
NW = NC * NS
L = 16

B = 16384
D = 64
N = 1000000
BPW = B // NW
CHUNK = 128
NCHUNK = BPW // CHUNK

BLK = 32768
GRID = (N + BLK - 1) // BLK


def _tc_scores_body(mtab_ref, utab_ref, w_ref, b_ref, om_ref, ou_ref):
    wm = w_ref[:D, :]
    wu = w_ref[D:, :]
    om_ref[...] = jnp.sum(mtab_ref[...] * wm, axis=0) + b_ref[0, 0]
    ou_ref[...] = jnp.sum(utab_ref[...] * wu, axis=0)


_tc_scores = pl.pallas_call(
    _tc_scores_body,
    grid=(GRID,),
    in_specs=[
        pl.BlockSpec((D, BLK), lambda g: (0, g)),
        pl.BlockSpec((D, BLK), lambda g: (0, g)),
        pl.BlockSpec((2 * D, 1), lambda g: (0, 0)),
        pl.BlockSpec((1, 1), lambda g: (0, 0)),
    ],
    out_specs=[
        pl.BlockSpec((BLK,), lambda g: (g,)),
        pl.BlockSpec((BLK,), lambda g: (g,)),
    ],
    out_shape=[
        jax.ShapeDtypeStruct((N,), jnp.float32),
        jax.ShapeDtypeStruct((N,), jnp.float32),
    ],
)


_mesh = plsc.VectorSubcoreMesh(
    core_axis_name="c", subcore_axis_name="s", num_cores=NC, num_subcores=NS
)


@functools.partial(
    pl.kernel,
    out_type=jax.ShapeDtypeStruct((NW, BPW), jnp.float32),
    mesh=_mesh,
    compiler_params=pltpu.CompilerParams(
        needs_layout_passes=False, use_tc_tiling_on_sc=False),
    scratch_types=[
        pltpu.VMEM((NCHUNK, CHUNK), jnp.int32),
        pltpu.VMEM((NCHUNK, CHUNK), jnp.int32),
        pltpu.VMEM((BPW,), jnp.float32),
        pltpu.VMEM((BPW,), jnp.float32),
        pltpu.SemaphoreType.DMA,
    ],
)
def _sc_combine(movies_hbm, users_hbm, ms_hbm, us_hbm,
                out_hbm, midx, uidx, gm, gu, sem):
    wid = lax.axis_index("s") * NC + lax.axis_index("c")

    pltpu.sync_copy(movies_hbm.at[wid], midx)
    pltpu.sync_copy(users_hbm.at[wid], uidx)

    descs = []
    for j in range(NCHUNK):
        descs.append(pltpu.async_copy(
            ms_hbm.at[midx.at[j]], gm.at[pl.ds(j * CHUNK, CHUNK)], sem))
        descs.append(pltpu.async_copy(
            us_hbm.at[uidx.at[j]], gu.at[pl.ds(j * CHUNK, CHUNK)], sem))
    for dsc in descs:
        dsc.wait()

    def group(g, carry):
        sl = pl.ds(g * L, L)
        gm[sl] = gm[sl] + gu[sl]
        return carry

    lax.fori_loop(0, BPW // L, group, 0)

    pltpu.sync_copy(gm, out_hbm.at[wid])


def kernel(users, movies, movie_table, user_table, head_w, head_b):
    movies_r = movies.reshape(NW, NCHUNK, CHUNK)
    users_r = users.reshape(NW, NCHUNK, CHUNK)
    b = head_b.reshape(1, 1)
    ms, us = _tc_scores(movie_table.T, user_table.T, head_w, b)
    out = _sc_combine(movies_r, users_r, ms, us)
    return out.reshape(B, 1)
